# baseline (speedup 1.0000x reference)
import functools

import jax
import jax.numpy as jnp
from jax import lax
from jax.experimental import pallas as pl
from jax.experimental.pallas import tpu as pltpu
from jax.experimental.pallas import tpu_sc as plsc

B = 16384
D = 32
W = 128
CHUNK = 128
NC, NS = 2, 16
NW = NC * NS
BPW = B // NW
NCHUNK = BPW // CHUNK
LBK = (13, 13, 7)


def _repack_body(x0_ref, x1_ref, x2_ref, x3_ref, o_ref):
    eye = (lax.broadcasted_iota(jnp.int32, (W, W), 0)
           == lax.broadcasted_iota(jnp.int32, (W, W), 1)).astype(jnp.float32)
    x_cat = jnp.concatenate(
        [x0_ref[...], x1_ref[...], x2_ref[...], x3_ref[...]], axis=0)
    o_ref[...] = lax.dot_general(x_cat, eye, (((0,), (0,)), ((), ())),
                                 preferred_element_type=jnp.float32)


def _repack(table_t, lb):
    d, v = table_t.shape
    nblk = -(-v // lb)
    steps = -(-v // (4 * lb))
    specs = [
        pl.BlockSpec((d, lb), functools.partial(
            lambda g, i: (0, jnp.minimum(4 * i + g, nblk - 1)), g))
        for g in range(4)
    ]
    return pl.pallas_call(
        _repack_body,
        grid=(steps,),
        in_specs=specs,
        out_specs=pl.BlockSpec((lb, W), lambda i: (i, 0)),
        out_shape=jax.ShapeDtypeStruct((steps * lb, W), jnp.float32),
    )(table_t, table_t, table_t, table_t)


@functools.lru_cache(maxsize=None)
def _make_gather(ntab, nrows, ks):
    mesh = plsc.VectorSubcoreMesh(core_axis_name="c", subcore_axis_name="s",
                                  num_cores=NC, num_subcores=NS)

    @functools.partial(
        pl.kernel,
        mesh=mesh,
        out_type=[jax.ShapeDtypeStruct((B, W), jnp.float32)
                  for _ in range(ntab)],
        scratch_types=(
            [pltpu.VMEM((NCHUNK, CHUNK), jnp.int32) for _ in range(ntab)]
            + [pltpu.VMEM((NCHUNK, CHUNK), jnp.int32) for _ in range(ntab)]
            + [pltpu.VMEM((CHUNK, W), jnp.float32) for _ in range(2 * ntab)]
            + [pltpu.SemaphoreType.DMA for _ in range(ntab)]
            + [pltpu.SemaphoreType.DMA for _ in range(ntab)]
        ),
    )
    def gather(*args):
        idx_hbms = args[:ntab]
        tables = args[ntab:2 * ntab]
        outs = args[2 * ntab:3 * ntab]
        rest = args[3 * ntab:]
        idx_vs = rest[:ntab]
        gidxs = rest[ntab:2 * ntab]
        flat_bufs = rest[2 * ntab:4 * ntab]
        bufs = [(flat_bufs[2 * t], flat_bufs[2 * t + 1])
                for t in range(ntab)]
        gsems = rest[4 * ntab:5 * ntab]
        wsems = rest[5 * ntab:6 * ntab]

        wid = lax.axis_index("s") * NC + lax.axis_index("c")
        base = wid * BPW
        crow = wid * NCHUNK

        for t in range(ntab):
            pltpu.sync_copy(idx_hbms[t].at[pl.ds(crow, NCHUNK)], idx_vs[t])
        for t in range(ntab):
            k = ks[t]
            for r in range(NCHUNK):
                for g in range(CHUNK // 16):
                    sl = pl.ds(g * 16, 16)
                    v = idx_vs[t][r, sl]
                    hi = lax.shift_left(
                        lax.shift_right_logical(v, k + 2), k)
                    gidxs[t][r, sl] = hi | (v & ((1 << k) - 1))

        gcp = {}
        wcp = {}
        for t in range(ntab):
            gcp[(t, 0)] = pltpu.async_copy(
                tables[t].at[gidxs[t].at[0]], bufs[t][0], gsems[t])
        for t in range(ntab):
            gcp[(t, 1)] = pltpu.async_copy(
                tables[t].at[gidxs[t].at[1]], bufs[t][1], gsems[t])
        for c in range(NCHUNK):
            for t in range(ntab):
                gcp[(t, c)].wait()
                wcp[(t, c)] = pltpu.async_copy(
                    bufs[t][c & 1],
                    outs[t].at[pl.ds(base + c * CHUNK, CHUNK)],
                    wsems[t])
            for t in range(ntab):
                if c + 2 < NCHUNK:
                    wcp[(t, c)].wait()
                    gcp[(t, c + 2)] = pltpu.async_copy(
                        tables[t].at[gidxs[t].at[c + 2]], bufs[t][c & 1],
                        gsems[t])
        for c in range(max(0, NCHUNK - 2), NCHUNK):
            for t in range(ntab):
                wcp[(t, c)].wait()

    return gather


def _mlp_body(uw_ref, iw_ref, sw_ref, uix_ref, iix_ref, six_ref,
              w1_ref, b1_ref, w2_ref, b2_ref, w3_ref, b3_ref, o_ref):
    bm = uw_ref.shape[0]
    lane_grp = lax.broadcasted_iota(jnp.int32, (bm, W), 1) >> 5

    def select(wide_ref, ix_ref, k):
        g = (lax.shift_right_logical(ix_ref[...], k)) & 3
        m = (lane_grp == g).astype(jnp.float32)
        x = wide_ref[...] * m
        return (x[:, 0:D] + x[:, D:2 * D] + x[:, 2 * D:3 * D]
                + x[:, 3 * D:4 * D])

    u = select(uw_ref, uix_ref, LBK[0])
    i = select(iw_ref, iix_ref, LBK[1])
    s = select(sw_ref, six_ref, LBK[2])
    h = (jnp.dot(u, w1_ref[0:D, :], preferred_element_type=jnp.float32)
         + jnp.dot(i, w1_ref[D:2 * D, :], preferred_element_type=jnp.float32)
         + jnp.dot(s, w1_ref[2 * D:3 * D, :], preferred_element_type=jnp.float32)
         + b1_ref[...])
    h = jnp.maximum(h, 0.0)
    h = jnp.dot(h, w2_ref[...], preferred_element_type=jnp.float32) + b2_ref[...]
    h = jnp.maximum(h, 0.0)
    o = jnp.dot(h, w3_ref[...], preferred_element_type=jnp.float32) + b3_ref[...]
    o_ref[...] = 1.0 / (1.0 + jnp.exp(-o))


def _mlp(u_wide, i_wide, s_wide, uix, iix, six, W1, b1, W2, b2, W3, b3,
         bm=4096):
    grid = (B // bm,)
    return pl.pallas_call(
        _mlp_body,
        grid=grid,
        in_specs=[
            pl.BlockSpec((bm, W), lambda i: (i, 0)),
            pl.BlockSpec((bm, W), lambda i: (i, 0)),
            pl.BlockSpec((bm, W), lambda i: (i, 0)),
            pl.BlockSpec((bm, 1), lambda i: (i, 0)),
            pl.BlockSpec((bm, 1), lambda i: (i, 0)),
            pl.BlockSpec((bm, 1), lambda i: (i, 0)),
            pl.BlockSpec((3 * D, 256), lambda i: (0, 0)),
            pl.BlockSpec((1, 256), lambda i: (0, 0)),
            pl.BlockSpec((256, 128), lambda i: (0, 0)),
            pl.BlockSpec((1, 128), lambda i: (0, 0)),
            pl.BlockSpec((128, 1), lambda i: (0, 0)),
            pl.BlockSpec((1, 1), lambda i: (0, 0)),
        ],
        out_specs=pl.BlockSpec((bm, 1), lambda i: (i, 0)),
        out_shape=jax.ShapeDtypeStruct((B, 1), jnp.float32),
    )(u_wide, i_wide, s_wide, uix, iix, six, W1, b1, W2, b2, W3, b3)


def kernel(user_input, pos_item_input, pos_item_subcategory_input,
           user_table, item_table, sub_table,
           W1, b1, W2, b2, W3, b3):
    uix = user_input.astype(jnp.int32)
    iix = pos_item_input.astype(jnp.int32)
    six = pos_item_subcategory_input.astype(jnp.int32)
    ut_w = _repack(user_table.T, 1 << LBK[0])
    st_w = _repack(sub_table.T, 1 << LBK[2])
    u_wide, s_wide = _make_gather(
        2, (ut_w.shape[0], st_w.shape[0]), (LBK[0], LBK[2]))(
        uix.reshape(B // CHUNK, CHUNK),
        six.reshape(B // CHUNK, CHUNK),
        ut_w, st_w)
    it_w = _repack(item_table.T, 1 << LBK[1])
    (i_wide,) = _make_gather(1, (it_w.shape[0],), (LBK[1],))(
        iix.reshape(B // CHUNK, CHUNK), it_w)
    return _mlp(u_wide, i_wide, s_wide,
                uix.reshape(B, 1), iix.reshape(B, 1), six.reshape(B, 1),
                W1, b1.reshape(1, 256), W2, b2.reshape(1, 128),
                W3, b3.reshape(1, 1))


# speedup vs baseline: 1.0088x; 1.0088x over previous
import functools

import jax
import jax.numpy as jnp
from jax import lax
from jax.experimental import pallas as pl
from jax.experimental.pallas import tpu as pltpu
from jax.experimental.pallas import tpu_sc as plsc

B = 16384
D = 32
W = 128
CHUNK = 128
NC, NS = 2, 16
NW = NC * NS
BPW = B // NW
NCHUNK = BPW // CHUNK
LBK = (14, 14, 7)


def _repack_body(x0_ref, x1_ref, x2_ref, x3_ref, o_ref):
    eye = (lax.broadcasted_iota(jnp.int32, (W, W), 0)
           == lax.broadcasted_iota(jnp.int32, (W, W), 1)).astype(jnp.float32)
    x_cat = jnp.concatenate(
        [x0_ref[...], x1_ref[...], x2_ref[...], x3_ref[...]], axis=0)
    o_ref[...] = lax.dot_general(x_cat, eye, (((0,), (0,)), ((), ())),
                                 preferred_element_type=jnp.float32)


def _repack(table_t, lb):
    d, v = table_t.shape
    nblk = -(-v // lb)
    steps = -(-v // (4 * lb))
    specs = [
        pl.BlockSpec((d, lb), functools.partial(
            lambda g, i: (0, jnp.minimum(4 * i + g, nblk - 1)), g))
        for g in range(4)
    ]
    return pl.pallas_call(
        _repack_body,
        grid=(steps,),
        in_specs=specs,
        out_specs=pl.BlockSpec((lb, W), lambda i: (i, 0)),
        out_shape=jax.ShapeDtypeStruct((steps * lb, W), jnp.float32),
    )(table_t, table_t, table_t, table_t)


@functools.lru_cache(maxsize=None)
def _make_gather(ntab, nrows, ks):
    mesh = plsc.VectorSubcoreMesh(core_axis_name="c", subcore_axis_name="s",
                                  num_cores=NC, num_subcores=NS)

    @functools.partial(
        pl.kernel,
        mesh=mesh,
        out_type=[jax.ShapeDtypeStruct((B, W), jnp.float32)
                  for _ in range(ntab)],
        scratch_types=(
            [pltpu.VMEM((NCHUNK, CHUNK), jnp.int32) for _ in range(ntab)]
            + [pltpu.VMEM((NCHUNK, CHUNK), jnp.int32) for _ in range(ntab)]
            + [pltpu.VMEM((CHUNK, W), jnp.float32) for _ in range(2 * ntab)]
            + [pltpu.SemaphoreType.DMA for _ in range(ntab)]
            + [pltpu.SemaphoreType.DMA for _ in range(ntab)]
        ),
    )
    def gather(*args):
        idx_hbms = args[:ntab]
        tables = args[ntab:2 * ntab]
        outs = args[2 * ntab:3 * ntab]
        rest = args[3 * ntab:]
        idx_vs = rest[:ntab]
        gidxs = rest[ntab:2 * ntab]
        flat_bufs = rest[2 * ntab:4 * ntab]
        bufs = [(flat_bufs[2 * t], flat_bufs[2 * t + 1])
                for t in range(ntab)]
        gsems = rest[4 * ntab:5 * ntab]
        wsems = rest[5 * ntab:6 * ntab]

        wid = lax.axis_index("s") * NC + lax.axis_index("c")
        base = wid * BPW
        crow = wid * NCHUNK

        for t in range(ntab):
            pltpu.sync_copy(idx_hbms[t].at[pl.ds(crow, NCHUNK)], idx_vs[t])
        for t in range(ntab):
            k = ks[t]
            for r in range(NCHUNK):
                for g in range(CHUNK // 16):
                    sl = pl.ds(g * 16, 16)
                    v = idx_vs[t][r, sl]
                    hi = lax.shift_left(
                        lax.shift_right_logical(v, k + 2), k)
                    gidxs[t][r, sl] = hi | (v & ((1 << k) - 1))

        gcp = {}
        wcp = {}
        for t in range(ntab):
            gcp[(t, 0)] = pltpu.async_copy(
                tables[t].at[gidxs[t].at[0]], bufs[t][0], gsems[t])
        for t in range(ntab):
            gcp[(t, 1)] = pltpu.async_copy(
                tables[t].at[gidxs[t].at[1]], bufs[t][1], gsems[t])
        for c in range(NCHUNK):
            for t in range(ntab):
                gcp[(t, c)].wait()
                wcp[(t, c)] = pltpu.async_copy(
                    bufs[t][c & 1],
                    outs[t].at[pl.ds(base + c * CHUNK, CHUNK)],
                    wsems[t])
            for t in range(ntab):
                if c + 2 < NCHUNK:
                    wcp[(t, c)].wait()
                    gcp[(t, c + 2)] = pltpu.async_copy(
                        tables[t].at[gidxs[t].at[c + 2]], bufs[t][c & 1],
                        gsems[t])
        for c in range(max(0, NCHUNK - 2), NCHUNK):
            for t in range(ntab):
                wcp[(t, c)].wait()

    return gather


def _mlp_body(uw_ref, iw_ref, sw_ref, uix_ref, iix_ref, six_ref,
              w1_ref, b1_ref, w2_ref, b2_ref, w3_ref, b3_ref, o_ref):
    bm = uw_ref.shape[0]
    lane_grp = lax.broadcasted_iota(jnp.int32, (bm, W), 1) >> 5

    def select(wide_ref, ix_ref, k):
        g = (lax.shift_right_logical(ix_ref[...], k)) & 3
        m = (lane_grp == g).astype(jnp.float32)
        x = wide_ref[...] * m
        return (x[:, 0:D] + x[:, D:2 * D] + x[:, 2 * D:3 * D]
                + x[:, 3 * D:4 * D])

    u = select(uw_ref, uix_ref, LBK[0])
    i = select(iw_ref, iix_ref, LBK[1])
    s = select(sw_ref, six_ref, LBK[2])
    h = (jnp.dot(u, w1_ref[0:D, :], preferred_element_type=jnp.float32)
         + jnp.dot(i, w1_ref[D:2 * D, :], preferred_element_type=jnp.float32)
         + jnp.dot(s, w1_ref[2 * D:3 * D, :], preferred_element_type=jnp.float32)
         + b1_ref[...])
    h = jnp.maximum(h, 0.0)
    h = jnp.dot(h, w2_ref[...], preferred_element_type=jnp.float32) + b2_ref[...]
    h = jnp.maximum(h, 0.0)
    o = jnp.dot(h, w3_ref[...], preferred_element_type=jnp.float32) + b3_ref[...]
    o_ref[...] = 1.0 / (1.0 + jnp.exp(-o))


def _mlp(u_wide, i_wide, s_wide, uix, iix, six, W1, b1, W2, b2, W3, b3,
         bm=4096):
    grid = (B // bm,)
    return pl.pallas_call(
        _mlp_body,
        grid=grid,
        in_specs=[
            pl.BlockSpec((bm, W), lambda i: (i, 0)),
            pl.BlockSpec((bm, W), lambda i: (i, 0)),
            pl.BlockSpec((bm, W), lambda i: (i, 0)),
            pl.BlockSpec((bm, 1), lambda i: (i, 0)),
            pl.BlockSpec((bm, 1), lambda i: (i, 0)),
            pl.BlockSpec((bm, 1), lambda i: (i, 0)),
            pl.BlockSpec((3 * D, 256), lambda i: (0, 0)),
            pl.BlockSpec((1, 256), lambda i: (0, 0)),
            pl.BlockSpec((256, 128), lambda i: (0, 0)),
            pl.BlockSpec((1, 128), lambda i: (0, 0)),
            pl.BlockSpec((128, 1), lambda i: (0, 0)),
            pl.BlockSpec((1, 1), lambda i: (0, 0)),
        ],
        out_specs=pl.BlockSpec((bm, 1), lambda i: (i, 0)),
        out_shape=jax.ShapeDtypeStruct((B, 1), jnp.float32),
    )(u_wide, i_wide, s_wide, uix, iix, six, W1, b1, W2, b2, W3, b3)


def kernel(user_input, pos_item_input, pos_item_subcategory_input,
           user_table, item_table, sub_table,
           W1, b1, W2, b2, W3, b3):
    uix = user_input.astype(jnp.int32)
    iix = pos_item_input.astype(jnp.int32)
    six = pos_item_subcategory_input.astype(jnp.int32)
    ut_w = _repack(user_table.T, 1 << LBK[0])
    st_w = _repack(sub_table.T, 1 << LBK[2])
    u_wide, s_wide = _make_gather(
        2, (ut_w.shape[0], st_w.shape[0]), (LBK[0], LBK[2]))(
        uix.reshape(B // CHUNK, CHUNK),
        six.reshape(B // CHUNK, CHUNK),
        ut_w, st_w)
    it_w = _repack(item_table.T, 1 << LBK[1])
    (i_wide,) = _make_gather(1, (it_w.shape[0],), (LBK[1],))(
        iix.reshape(B // CHUNK, CHUNK), it_w)
    return _mlp(u_wide, i_wide, s_wide,
                uix.reshape(B, 1), iix.reshape(B, 1), six.reshape(B, 1),
                W1, b1.reshape(1, 256), W2, b2.reshape(1, 128),
                W3, b3.reshape(1, 1))
